# Initial kernel scaffold; baseline (speedup 1.0000x reference)
import jax
import jax.numpy as jnp
from jax.experimental import pallas as pl


def kernel(src, W0, W1, block_assignment, local_assignment):
    raise NotImplementedError("write your pallas kernel here")


# speedup vs baseline: 7.7513x; 7.7513x over previous
import functools

import jax
import jax.numpy as jnp
from jax import lax
from jax.experimental import pallas as pl
from jax.experimental.pallas import tpu as pltpu
from jax.experimental.pallas import tpu_sc as plsc

VOCAB = 100
BLOCK = 50
DIM = 64

_NC = 2
_NS = 16
_L = 16
_NW = _NC * _NS
_CHUNK = 128


def _make_sc_gather(n_tok):
    t_per_w = n_tok // _NW
    n_chunk = t_per_w // _CHUNK
    vec_per_chunk = _CHUNK // _L

    mesh = plsc.VectorSubcoreMesh(core_axis_name="c", subcore_axis_name="s")

    @functools.partial(
        pl.kernel,
        mesh=mesh,
        out_type=jax.ShapeDtypeStruct((n_tok, DIM), jnp.float32),
        compiler_params=pltpu.CompilerParams(
            needs_layout_passes=False, use_tc_tiling_on_sc=False),
        scratch_types=[
            pltpu.VMEM((t_per_w,), jnp.int32),
            pltpu.VMEM((VOCAB,), jnp.int32),
            pltpu.VMEM((VOCAB,), jnp.int32),
            pltpu.VMEM((n_chunk, _CHUNK), jnp.int32),
            pltpu.VMEM((t_per_w, DIM), jnp.float32),
            pltpu.SemaphoreType.DMA,
        ],
    )
    def k(src_hbm, ba_hbm, la_hbm, table_hbm, out_hbm,
          src_v, ba_v, la_v, idx_v, rows_v, sem):
        wid = lax.axis_index("s") * _NC + lax.axis_index("c")
        base = wid * t_per_w
        pltpu.sync_copy(src_hbm.at[pl.ds(base, t_per_w)], src_v)
        pltpu.sync_copy(ba_hbm, ba_v)
        pltpu.sync_copy(la_hbm, la_v)
        copies = []
        for j in range(n_chunk):
            for g in range(vec_per_chunk):
                i = j * vec_per_chunk + g
                s = src_v[pl.ds(i * _L, _L)]
                b = plsc.load_gather(ba_v, [s])
                l = plsc.load_gather(la_v, [s])
                idx_v[j, pl.ds(g * _L, _L)] = b * BLOCK + l
            copies.append(
                pltpu.async_copy(table_hbm.at[idx_v.at[j]],
                                 rows_v.at[pl.ds(j * _CHUNK, _CHUNK)], sem))
        for c in copies:
            c.wait()
        pltpu.sync_copy(rows_v, out_hbm.at[pl.ds(base, t_per_w)])

    return k


def kernel(src, W0, W1, block_assignment, local_assignment):
    n_tok = src.shape[0] * src.shape[1]
    table = jnp.concatenate([W0, W1], axis=0)
    out = _make_sc_gather(n_tok)(
        src.reshape(n_tok), block_assignment, local_assignment, table)
    return out.reshape(src.shape + (DIM,))
